# baseline (speedup 1.0000x reference)
import math

import jax
import jax.numpy as jnp
from jax import lax
from jax.experimental import pallas as pl
from jax.experimental.pallas import tpu as pltpu
from jax.experimental.pallas import tpu_sc as plsc

D_MODEL = 64
SCALE = math.sqrt(D_MODEL)
NC, NS = 2, 16
NW = NC * NS
GRP = 128
LANES = 16
NBLK = D_MODEL // LANES


def _emb_body(idx_hbm, table2_hbm, out_hbm, xstage, ih0, ih1, g0, g1,
              st0, st1, gs0, gs1, os0, os1):
    wid = lax.axis_index("s") * NC + lax.axis_index("c")
    n_idx_rows = idx_hbm.shape[0]
    per_w = n_idx_rows // NW
    ibase = pl.multiple_of(wid * per_w, 8)
    obase = wid * per_w * (GRP // 2)

    pltpu.sync_copy(idx_hbm.at[pl.ds(ibase, per_w), :], xstage)

    def compute_ih(c, ih):
        for k in range(GRP // LANES):
            sl = pl.ds(k * LANES, LANES)
            ih[0, sl] = lax.shift_right_logical(xstage[c, sl], 1)

    def fire_gather(ih, gbuf, gsem):
        pltpu.async_copy(table2_hbm.at[ih.at[0]], gbuf, gsem)

    def wait_gather(gbuf, gsem):
        pltpu.make_async_copy(
            table2_hbm.at[pl.ds(0, GRP), :], gbuf, gsem).wait()

    def select_scale(c, gbuf, stg):
        def group(g16, carry):
            hvec = (xstage[c, pl.ds(g16 * LANES, LANES)] & 1) * D_MODEL
            for lane in range(LANES):
                r = g16 * LANES + lane
                i2 = g16 * (LANES // 2) + lane // 2
                h64 = hvec[lane]
                for cb in range(NBLK):
                    src = pl.ds(h64 + cb * LANES, LANES)
                    dst = pl.ds((lane % 2) * D_MODEL + cb * LANES, LANES)
                    stg[i2, dst] = gbuf[r, src] * SCALE
            return carry

        lax.fori_loop(0, GRP // LANES, group, 0)

    def fire_out(c, stg, osem):
        rb = obase + c * (GRP // 2)
        pltpu.async_copy(stg, out_hbm.at[pl.ds(rb, GRP // 2), :], osem)

    def wait_out(stg, osem):
        pltpu.make_async_copy(
            stg, out_hbm.at[pl.ds(0, GRP // 2), :], osem).wait()

    n_steps = per_w // 2
    compute_ih(0, ih0)
    fire_gather(ih0, g0, gs0)

    def step(s, carry):
        c0 = 2 * s

        @pl.when(s > 0)
        def _():
            wait_out(st1, os1)

        compute_ih(c0 + 1, ih1)
        fire_gather(ih1, g1, gs1)
        wait_gather(g0, gs0)
        select_scale(c0, g0, st0)
        fire_out(c0, st0, os0)

        @pl.when(s < n_steps - 1)
        def _():
            wait_out(st0, os0)
            compute_ih(c0 + 2, ih0)
            fire_gather(ih0, g0, gs0)

        wait_gather(g1, gs1)
        select_scale(c0 + 1, g1, st1)
        fire_out(c0 + 1, st1, os1)
        return carry

    lax.fori_loop(0, n_steps, step, 0)
    wait_out(st0, os0)
    wait_out(st1, os1)


def kernel(X, table):
    n = X.shape[0] * X.shape[1]
    idx2d = X.reshape(n // GRP, GRP).astype(jnp.int32)
    table2 = table.reshape(table.shape[0] // 2, 2 * D_MODEL)
    mesh = plsc.VectorSubcoreMesh(core_axis_name="c", subcore_axis_name="s")
    run = pl.kernel(
        _emb_body,
        out_type=jax.ShapeDtypeStruct((n // 2, 2 * D_MODEL), jnp.float32),
        mesh=mesh,
        compiler_params=pltpu.CompilerParams(use_tc_tiling_on_sc=True),
        scratch_types=[
            pltpu.VMEM((n // GRP // NW, GRP), jnp.int32),
            pltpu.VMEM((1, GRP), jnp.int32),
            pltpu.VMEM((1, GRP), jnp.int32),
            pltpu.VMEM((GRP, GRP), jnp.float32),
            pltpu.VMEM((GRP, GRP), jnp.float32),
            pltpu.VMEM((GRP // 2, GRP), jnp.float32),
            pltpu.VMEM((GRP // 2, GRP), jnp.float32),
            pltpu.SemaphoreType.DMA,
            pltpu.SemaphoreType.DMA,
            pltpu.SemaphoreType.DMA,
            pltpu.SemaphoreType.DMA,
        ],
    )
    out2 = run(idx2d, table2)
    return out2.reshape(X.shape[0], X.shape[1], D_MODEL)


# speedup vs baseline: 1.6407x; 1.6407x over previous
import math

import jax
import jax.numpy as jnp
from jax import lax
from jax.experimental import pallas as pl
from jax.experimental.pallas import tpu as pltpu
from jax.experimental.pallas import tpu_sc as plsc

D_MODEL = 64
SCALE = math.sqrt(D_MODEL)
NC, NS = 2, 16
NW = NC * NS
GRP = 128
PADW = 2 * D_MODEL
LANES = 16
NBLK = D_MODEL // LANES


def _emb_body(idx_hbm, tpad_hbm, out_hbm, xstage, g0, g1, h0, h1, gs0, gs1,
              os0, os1):
    wid = lax.axis_index("s") * NC + lax.axis_index("c")
    n_idx_rows = idx_hbm.shape[0]
    per_w = n_idx_rows // NW
    ibase = pl.multiple_of(wid * per_w, 8)
    obase = wid * per_w * GRP

    pltpu.sync_copy(idx_hbm.at[pl.ds(ibase, per_w), :], xstage)

    def fire_gather(c, gbuf, gsem):
        pltpu.async_copy(tpad_hbm.at[xstage.at[c]], gbuf, gsem)

    def wait_gather(gbuf, gsem):
        pltpu.make_async_copy(
            tpad_hbm.at[pl.ds(0, GRP), :], gbuf, gsem).wait()

    def scale(gbuf, hbuf):
        def row(r, carry):
            for cb in range(NBLK):
                sl = pl.ds(cb * LANES, LANES)
                hbuf[r, sl] = gbuf[r, sl] * SCALE
            return carry

        lax.fori_loop(0, GRP, row, 0)

    def fire_out(c, hbuf, osem):
        rb = obase + c * GRP
        pltpu.async_copy(hbuf, out_hbm.at[pl.ds(rb, GRP), :], osem)

    def wait_out(hbuf, osem):
        pltpu.make_async_copy(
            hbuf, out_hbm.at[pl.ds(0, GRP), :], osem).wait()

    n_steps = per_w // 2
    fire_gather(0, g0, gs0)
    fire_gather(1, g1, gs1)

    def step(s, carry):
        c0 = 2 * s

        @pl.when(s > 0)
        def _():
            wait_out(h0, os0)

        wait_gather(g0, gs0)
        scale(g0, h0)
        fire_out(c0, h0, os0)

        @pl.when(s < n_steps - 1)
        def _():
            fire_gather(c0 + 2, g0, gs0)

        @pl.when(s > 0)
        def _():
            wait_out(h1, os1)

        wait_gather(g1, gs1)
        scale(g1, h1)
        fire_out(c0 + 1, h1, os1)

        @pl.when(s < n_steps - 1)
        def _():
            fire_gather(c0 + 3, g1, gs1)

        return carry

    lax.fori_loop(0, n_steps, step, 0)
    wait_out(h0, os0)
    wait_out(h1, os1)


def kernel(X, table):
    n = X.shape[0] * X.shape[1]
    idx2d = X.reshape(n // GRP, GRP).astype(jnp.int32)
    tpad = jnp.pad(table, ((0, 0), (0, PADW - D_MODEL)))
    mesh = plsc.VectorSubcoreMesh(core_axis_name="c", subcore_axis_name="s")
    run = pl.kernel(
        _emb_body,
        out_type=jax.ShapeDtypeStruct((n, D_MODEL), jnp.float32),
        mesh=mesh,
        compiler_params=pltpu.CompilerParams(use_tc_tiling_on_sc=True),
        scratch_types=[
            pltpu.VMEM((n // GRP // NW, GRP), jnp.int32),
            pltpu.VMEM((GRP, PADW), jnp.float32),
            pltpu.VMEM((GRP, PADW), jnp.float32),
            pltpu.VMEM((GRP, D_MODEL), jnp.float32),
            pltpu.VMEM((GRP, D_MODEL), jnp.float32),
            pltpu.SemaphoreType.DMA,
            pltpu.SemaphoreType.DMA,
            pltpu.SemaphoreType.DMA,
            pltpu.SemaphoreType.DMA,
        ],
    )
    out = run(idx2d, tpad)
    return out.reshape(X.shape[0], X.shape[1], D_MODEL)
